# baseline (speedup 1.0000x reference)
import jax
import jax.numpy as jnp
from jax.experimental import pallas as pl
from jax.experimental.pallas import tpu as pltpu

_TEMPERATURE = 1.0
_BLOCK_COLS = 1024
_SLOTS = 6


def _scale_stream_kernel(x_ref, o_ref, buf, in_sems, out_sems):
    n = x_ref.shape[1] // _BLOCK_COLS

    def in_copy(i):
        return pltpu.make_async_copy(
            x_ref.at[:, pl.ds(i * _BLOCK_COLS, _BLOCK_COLS)],
            buf.at[i % _SLOTS],
            in_sems.at[i % _SLOTS],
        )

    def out_copy(i):
        return pltpu.make_async_copy(
            buf.at[i % _SLOTS],
            o_ref.at[:, pl.ds(i * _BLOCK_COLS, _BLOCK_COLS)],
            out_sems.at[i % _SLOTS],
        )

    for i in range(min(_SLOTS - 1, n)):
        in_copy(i).start()
    for i in range(n):
        in_copy(i).wait()
        out_copy(i).start()
        nxt = i + _SLOTS - 1
        if nxt < n:
            prev = nxt - _SLOTS
            if prev >= 0:
                out_copy(prev).wait()
            in_copy(nxt).start()
    for i in range(max(0, n - _SLOTS), n):
        out_copy(i).wait()


def kernel(input, labels):
    rows, cols = input.shape
    xt = input.T
    yt = pl.pallas_call(
        _scale_stream_kernel,
        in_specs=[pl.BlockSpec(memory_space=pltpu.MemorySpace.HBM)],
        out_specs=pl.BlockSpec(memory_space=pltpu.MemorySpace.HBM),
        out_shape=jax.ShapeDtypeStruct((cols, rows), input.dtype),
        scratch_shapes=[
            pltpu.VMEM((_SLOTS, cols, _BLOCK_COLS), jnp.float32),
            pltpu.SemaphoreType.DMA((_SLOTS,)),
            pltpu.SemaphoreType.DMA((_SLOTS,)),
        ],
    )(xt)
    return yt.T


# speedup vs baseline: 1.0063x; 1.0063x over previous
import jax
import jax.numpy as jnp
from jax.experimental import pallas as pl
from jax.experimental.pallas import tpu as pltpu

_TEMPERATURE = 1.0
_BLOCK_COLS = 2048
_SLOTS = 5


def _scale_stream_kernel(x_ref, o_ref, buf, in_sems, out_sems):
    n = x_ref.shape[1] // _BLOCK_COLS

    def in_copy(i):
        return pltpu.make_async_copy(
            x_ref.at[:, pl.ds(i * _BLOCK_COLS, _BLOCK_COLS)],
            buf.at[i % _SLOTS],
            in_sems.at[i % _SLOTS],
        )

    def out_copy(i):
        return pltpu.make_async_copy(
            buf.at[i % _SLOTS],
            o_ref.at[:, pl.ds(i * _BLOCK_COLS, _BLOCK_COLS)],
            out_sems.at[i % _SLOTS],
        )

    for i in range(min(_SLOTS - 1, n)):
        in_copy(i).start()
    for i in range(n):
        in_copy(i).wait()
        out_copy(i).start()
        nxt = i + _SLOTS - 1
        if nxt < n:
            prev = nxt - _SLOTS
            if prev >= 0:
                out_copy(prev).wait()
            in_copy(nxt).start()
    for i in range(max(0, n - _SLOTS), n):
        out_copy(i).wait()


def kernel(input, labels):
    rows, cols = input.shape
    xt = input.T
    yt = pl.pallas_call(
        _scale_stream_kernel,
        in_specs=[pl.BlockSpec(memory_space=pltpu.MemorySpace.HBM)],
        out_specs=pl.BlockSpec(memory_space=pltpu.MemorySpace.HBM),
        out_shape=jax.ShapeDtypeStruct((cols, rows), input.dtype),
        scratch_shapes=[
            pltpu.VMEM((_SLOTS, cols, _BLOCK_COLS), jnp.float32),
            pltpu.SemaphoreType.DMA((_SLOTS,)),
            pltpu.SemaphoreType.DMA((_SLOTS,)),
        ],
    )(xt)
    return yt.T


# speedup vs baseline: 1.0107x; 1.0043x over previous
import jax
import jax.numpy as jnp
from jax.experimental import pallas as pl
from jax.experimental.pallas import tpu as pltpu

_TEMPERATURE = 1.0
_BLOCK_COLS = 2048
_SLOTS = 4


def _scale_stream_kernel(x_ref, o_ref, buf, in_sems, out_sems):
    n = x_ref.shape[1] // _BLOCK_COLS

    def in_copy(i):
        return pltpu.make_async_copy(
            x_ref.at[:, pl.ds(i * _BLOCK_COLS, _BLOCK_COLS)],
            buf.at[i % _SLOTS],
            in_sems.at[i % _SLOTS],
        )

    def out_copy(i):
        return pltpu.make_async_copy(
            buf.at[i % _SLOTS],
            o_ref.at[:, pl.ds(i * _BLOCK_COLS, _BLOCK_COLS)],
            out_sems.at[i % _SLOTS],
        )

    for i in range(min(_SLOTS - 1, n)):
        in_copy(i).start()
    for i in range(n):
        in_copy(i).wait()
        out_copy(i).start()
        nxt = i + _SLOTS - 1
        if nxt < n:
            prev = nxt - _SLOTS
            if prev >= 0:
                out_copy(prev).wait()
            in_copy(nxt).start()
    for i in range(max(0, n - _SLOTS), n):
        out_copy(i).wait()


def kernel(input, labels):
    rows, cols = input.shape
    xt = input.T
    yt = pl.pallas_call(
        _scale_stream_kernel,
        in_specs=[pl.BlockSpec(memory_space=pltpu.MemorySpace.HBM)],
        out_specs=pl.BlockSpec(memory_space=pltpu.MemorySpace.HBM),
        out_shape=jax.ShapeDtypeStruct((cols, rows), input.dtype),
        scratch_shapes=[
            pltpu.VMEM((_SLOTS, cols, _BLOCK_COLS), jnp.float32),
            pltpu.SemaphoreType.DMA((_SLOTS,)),
            pltpu.SemaphoreType.DMA((_SLOTS,)),
        ],
    )(xt)
    return yt.T
